# baseline (speedup 1.0000x reference)
import functools

import jax
import jax.numpy as jnp
from jax import lax
from jax.experimental import pallas as pl
from jax.experimental.pallas import tpu as pltpu
from jax.experimental.pallas import tpu_sc as plsc

N = 10000
E = 320000
D = 128
NPAD = 10240
NTILE = 16
RPT = NPAD // NTILE
EB = 112
BPT = 180
KGRP = 10
EPAD = NTILE * BPT * EB
EPT = BPT * EB
PIECE = 32


def _sc_mesh():
    return plsc.VectorSubcoreMesh(core_axis_name="c", subcore_axis_name="s")


def _zero_piece(piece):
    z16 = jnp.zeros((16,), jnp.float32)

    def _zp(r, carry):
        for j in range(D // 16):
            piece[r, pl.ds(j * 16, 16)] = z16
        return carry
    lax.fori_loop(0, PIECE, _zp, 0)


def _run_dir(vals_hbm, g1d_hbm, s1d_hbm, acc, idxg2, idxs2, rows, sem,
             sid, dacc=None, ones_v=None):
    ebase = sid * EPT

    def group(g, carry):
        goff = ebase + g * (KGRP * EB)
        pltpu.sync_copy(g1d_hbm.at[pl.ds(goff, KGRP * EB)], idxg2)
        pltpu.sync_copy(s1d_hbm.at[pl.ds(goff, KGRP * EB)], idxs2)
        if dacc is not None:
            pltpu.sync_copy(ones_v, dacc.at[idxs2], add=True)
        for j in range(KGRP):
            pltpu.async_copy(
                vals_hbm.at[idxg2.at[pl.ds(j * EB, EB)]], rows, sem).wait()
            pltpu.sync_copy(rows,
                            acc.at[idxs2.at[pl.ds(j * EB, EB)]], add=True)
        return carry
    lax.fori_loop(0, BPT // KGRP, group, 0)


def _phase1_body(x_hbm, srcg_hbm, dstg_hbm, srcs_hbm, dsts_hbm,
                 sin_hbm, sout_hbm, din_hbm, dout_hbm,
                 idxg2, idxs2, rows, piece, degbuf, ones_v,
                 acc, dacc, sem):
    c = lax.axis_index("c")
    sid = lax.axis_index("s")
    base_r = sid * RPT
    z16 = jnp.zeros((16,), jnp.float32)

    _zero_piece(piece)

    def _zd(i, carry):
        degbuf[pl.ds(i * 16, 16)] = z16
        return carry
    lax.fori_loop(0, RPT // 16, _zd, 0)

    def _zo(i, carry):
        ones_v[pl.ds(i * 16, 16)] = jnp.ones((16,), jnp.float32)
        return carry
    lax.fori_loop(0, KGRP * EB // 16, _zo, 0)

    for k in range(RPT // PIECE):
        pltpu.sync_copy(piece, acc.at[pl.ds(base_r + k * PIECE, PIECE)])
    pltpu.sync_copy(degbuf, dacc.at[pl.ds(base_r, RPT)])
    plsc.subcore_barrier()

    args = (acc, idxg2, idxs2, rows, sem, sid)

    @pl.when(c == 0)
    def _():
        _run_dir(x_hbm, srcg_hbm, dsts_hbm, *args, dacc=dacc, ones_v=ones_v)

    @pl.when(c == 1)
    def _():
        _run_dir(x_hbm, dstg_hbm, srcs_hbm, *args, dacc=dacc, ones_v=ones_v)

    plsc.subcore_barrier()

    def finish(s_hbm, d_hbm):
        pltpu.sync_copy(dacc.at[pl.ds(base_r, RPT)], degbuf)
        pltpu.sync_copy(degbuf, d_hbm.at[pl.ds(base_r, RPT)])
        for k in range(RPT // PIECE):
            pltpu.sync_copy(acc.at[pl.ds(base_r + k * PIECE, PIECE)], piece)
            pltpu.sync_copy(piece, s_hbm.at[pl.ds(base_r + k * PIECE, PIECE)])

    @pl.when(c == 0)
    def _():
        finish(sin_hbm, din_hbm)

    @pl.when(c == 1)
    def _():
        finish(sout_hbm, dout_hbm)


_phase1 = functools.partial(
    pl.kernel,
    out_type=[
        jax.ShapeDtypeStruct((NPAD, D), jnp.float32),
        jax.ShapeDtypeStruct((NPAD, D), jnp.float32),
        jax.ShapeDtypeStruct((NPAD,), jnp.float32),
        jax.ShapeDtypeStruct((NPAD,), jnp.float32),
    ],
    mesh=_sc_mesh(),
    scratch_types=[
        pltpu.VMEM((KGRP * EB,), jnp.int32),
        pltpu.VMEM((KGRP * EB,), jnp.int32),
        pltpu.VMEM((EB, D), jnp.float32),
        pltpu.VMEM((PIECE, D), jnp.float32),
        pltpu.VMEM((RPT,), jnp.float32),
        pltpu.VMEM((KGRP * EB,), jnp.float32),
        pltpu.VMEM_SHARED((NPAD, D), jnp.float32),
        pltpu.VMEM_SHARED((NPAD,), jnp.float32),
        pltpu.SemaphoreType.DMA,
    ],
)(_phase1_body)


def _phase2_body(uin_hbm, uout_hbm, srcg_hbm, dstg_hbm, srcs_hbm, dsts_hbm,
                 tin_hbm, tout_hbm,
                 idxg2, idxs2, rows, piece,
                 acc, sem):
    c = lax.axis_index("c")
    sid = lax.axis_index("s")
    base_r = sid * RPT

    _zero_piece(piece)
    for k in range(RPT // PIECE):
        pltpu.sync_copy(piece, acc.at[pl.ds(base_r + k * PIECE, PIECE)])
    plsc.subcore_barrier()

    @pl.when(c == 0)
    def _():
        _run_dir(uin_hbm, srcg_hbm, dsts_hbm, acc, idxg2, idxs2, rows,
                 sem, sid)

    @pl.when(c == 1)
    def _():
        _run_dir(uout_hbm, dstg_hbm, srcs_hbm, acc, idxg2, idxs2, rows,
                 sem, sid)

    plsc.subcore_barrier()

    def finish(t_hbm):
        for k in range(RPT // PIECE):
            pltpu.sync_copy(acc.at[pl.ds(base_r + k * PIECE, PIECE)], piece)
            pltpu.sync_copy(piece, t_hbm.at[pl.ds(base_r + k * PIECE, PIECE)])

    @pl.when(c == 0)
    def _():
        finish(tin_hbm)

    @pl.when(c == 1)
    def _():
        finish(tout_hbm)


_phase2 = functools.partial(
    pl.kernel,
    out_type=[
        jax.ShapeDtypeStruct((NPAD, D), jnp.float32),
        jax.ShapeDtypeStruct((NPAD, D), jnp.float32),
    ],
    mesh=_sc_mesh(),
    scratch_types=[
        pltpu.VMEM((KGRP * EB,), jnp.int32),
        pltpu.VMEM((KGRP * EB,), jnp.int32),
        pltpu.VMEM((EB, D), jnp.float32),
        pltpu.VMEM((PIECE, D), jnp.float32),
        pltpu.VMEM_SHARED((NPAD, D), jnp.float32),
        pltpu.SemaphoreType.DMA,
    ],
)(_phase2_body)


def _fold_body(W1_ref, W2_ref, b1c_ref, kin_ref, kout_ref, cin_ref, cout_ref):
    W1 = W1_ref[...]
    Wc = jnp.concatenate(
        [W1[:, :D] + W1[:, D:2 * D], W1[:, 2 * D:3 * D] + W1[:, 3 * D:]],
        axis=1)
    W2in = W2_ref[:, :1024]
    W2out = W2_ref[:, 1024:]
    b1c = b1c_ref[...]
    kin_ref[...] = jnp.dot(W2in, Wc, preferred_element_type=jnp.float32)
    kout_ref[...] = jnp.dot(W2out, Wc, preferred_element_type=jnp.float32)
    cin_ref[...] = jnp.dot(W2in, b1c, preferred_element_type=jnp.float32)
    cout_ref[...] = jnp.dot(W2out, b1c, preferred_element_type=jnp.float32)


def _fold(W1, W2, b1c):
    return pl.pallas_call(
        _fold_body,
        out_shape=[
            jax.ShapeDtypeStruct((D, 2 * D), jnp.float32),
            jax.ShapeDtypeStruct((D, 2 * D), jnp.float32),
            jax.ShapeDtypeStruct((D, 1), jnp.float32),
            jax.ShapeDtypeStruct((D, 1), jnp.float32),
        ],
    )(W1, W2, b1c)


BLK = 1024


def _elu(v):
    return jnp.where(v > 0.0, v, jnp.exp(v) - 1.0)


def _proj_body(sin_ref, sout_ref, din_ref, dout_ref, min_ref, mout_ref,
               cin_ref, cout_ref, uin_ref, uout_ref):
    inv_in = 1.0 / jnp.maximum(din_ref[...], 1.0)
    inv_out = 1.0 / jnp.maximum(dout_ref[...], 1.0)
    f = jnp.concatenate(
        [_elu(sin_ref[...] * inv_in), _elu(sout_ref[...] * inv_out)],
        axis=1)
    uin_ref[...] = (
        jnp.dot(f, min_ref[...], preferred_element_type=jnp.float32)
        + cin_ref[...])
    uout_ref[...] = (
        jnp.dot(f, mout_ref[...], preferred_element_type=jnp.float32)
        + cout_ref[...])


def _proj(s_in, s_out, din_c, dout_c, Min, Mout, cin_r, cout_r):
    grid = (NPAD // BLK,)
    row_spec = pl.BlockSpec((BLK, D), lambda i: (i, 0))
    col_spec = pl.BlockSpec((BLK, 1), lambda i: (i, 0))
    full_spec = pl.BlockSpec((2 * D, D), lambda i: (0, 0))
    bias_spec = pl.BlockSpec((1, D), lambda i: (0, 0))
    return pl.pallas_call(
        _proj_body,
        grid=grid,
        in_specs=[row_spec, row_spec, col_spec, col_spec,
                  full_spec, full_spec, bias_spec, bias_spec],
        out_specs=[row_spec, row_spec],
        out_shape=[
            jax.ShapeDtypeStruct((NPAD, D), jnp.float32),
            jax.ShapeDtypeStruct((NPAD, D), jnp.float32),
        ],
    )(s_in, s_out, din_c, dout_c, Min, Mout, cin_r, cout_r)


def _final_body(tin_ref, tout_ref, din_ref, dout_ref, b2_ref, out_ref):
    inv_in = 1.0 / jnp.maximum(din_ref[...], 1.0)
    inv_out = 1.0 / jnp.maximum(dout_ref[...], 1.0)
    out_ref[...] = (tin_ref[...] * inv_in + tout_ref[...] * inv_out
                    + b2_ref[...])


def _final(t_in, t_out, din_c, dout_c, b2r):
    grid = (NPAD // BLK,)
    row_spec = pl.BlockSpec((BLK, D), lambda i: (i, 0))
    col_spec = pl.BlockSpec((BLK, 1), lambda i: (i, 0))
    bias_spec = pl.BlockSpec((1, D), lambda i: (0, 0))
    return pl.pallas_call(
        _final_body,
        grid=grid,
        in_specs=[row_spec, row_spec, col_spec, col_spec, bias_spec],
        out_specs=row_spec,
        out_shape=jax.ShapeDtypeStruct((NPAD, D), jnp.float32),
    )(t_in, t_out, din_c, dout_c, b2r)


@jax.jit
def kernel(feature, edge_index, W1, b1, W2, b2):
    npad_e = EPAD - E
    pad_g = jnp.zeros((npad_e,), jnp.int32)
    pad_s = N + (jnp.arange(npad_e, dtype=jnp.int32) % (NPAD - N))
    srcg = jnp.concatenate([edge_index[0], pad_g])
    dstg = jnp.concatenate([edge_index[1], pad_g])
    srcs = jnp.concatenate([edge_index[0], pad_s])
    dsts = jnp.concatenate([edge_index[1], pad_s])
    b1c = b1.reshape(-1, 1)
    kin, kout, cin, cout = _fold(W1, W2, b1c)
    s_in, s_out, din, dout = _phase1(feature, srcg, dstg, srcs, dsts)
    din_c = din.reshape(NPAD, 1)
    dout_c = dout.reshape(NPAD, 1)
    u_in, u_out = _proj(s_in, s_out, din_c, dout_c,
                        kin.T, kout.T, cin.T, cout.T)
    t_in, t_out = _phase2(u_in, u_out, srcg, dstg, srcs, dsts)
    out = _final(t_in, t_out, din_c, dout_c, b2.reshape(1, -1))
    return out[:N]


# speedup vs baseline: 1.2670x; 1.2670x over previous
import functools

import jax
import jax.numpy as jnp
from jax import lax
from jax.experimental import pallas as pl
from jax.experimental.pallas import tpu as pltpu
from jax.experimental.pallas import tpu_sc as plsc

N = 10000
E = 320000
D = 128
NPAD = 10240
NTILE = 16
RPT = NPAD // NTILE
EB = 112
BPT = 180
KGRP = 10
EPAD = NTILE * BPT * EB
EPT = BPT * EB
PIECE = 32


def _sc_mesh():
    return plsc.VectorSubcoreMesh(core_axis_name="c", subcore_axis_name="s")


def _zero_piece(piece):
    z16 = jnp.zeros((16,), jnp.float32)

    def _zp(r, carry):
        for j in range(D // 16):
            piece[r, pl.ds(j * 16, 16)] = z16
        return carry
    lax.fori_loop(0, PIECE, _zp, 0)


def _run_dir(vals_hbm, g1d_hbm, s1d_hbm, acc, idxg2, idxs2,
             rows0, sem0, rows1, sem1, sid, dacc=None, ones_v=None):
    ebase = sid * EPT
    rows = (rows0, rows1)
    sems = (sem0, sem1)

    def group(g, carry):
        goff = ebase + g * (KGRP * EB)
        pltpu.sync_copy(g1d_hbm.at[pl.ds(goff, KGRP * EB)], idxg2)
        pltpu.sync_copy(s1d_hbm.at[pl.ds(goff, KGRP * EB)], idxs2)
        if dacc is not None:
            pltpu.sync_copy(ones_v, dacc.at[idxs2], add=True)
        descr = pltpu.async_copy(
            vals_hbm.at[idxg2.at[pl.ds(0, EB)]], rows[0], sems[0])
        for j in range(KGRP):
            nxt = None
            if j + 1 < KGRP:
                nxt = pltpu.async_copy(
                    vals_hbm.at[idxg2.at[pl.ds((j + 1) * EB, EB)]],
                    rows[(j + 1) % 2], sems[(j + 1) % 2])
            descr.wait()
            pltpu.sync_copy(rows[j % 2],
                            acc.at[idxs2.at[pl.ds(j * EB, EB)]], add=True)
            descr = nxt
        return carry
    lax.fori_loop(0, BPT // KGRP, group, 0)


def _phase1_body(x_hbm, srcg_hbm, dstg_hbm, srcs_hbm, dsts_hbm,
                 sin_hbm, sout_hbm, din_hbm, dout_hbm,
                 idxg2, idxs2, rows0, rows1, piece, degbuf, ones_v,
                 acc, dacc, sem0, sem1):
    c = lax.axis_index("c")
    sid = lax.axis_index("s")
    base_r = sid * RPT
    z16 = jnp.zeros((16,), jnp.float32)

    _zero_piece(piece)

    def _zd(i, carry):
        degbuf[pl.ds(i * 16, 16)] = z16
        return carry
    lax.fori_loop(0, RPT // 16, _zd, 0)

    def _zo(i, carry):
        ones_v[pl.ds(i * 16, 16)] = jnp.ones((16,), jnp.float32)
        return carry
    lax.fori_loop(0, KGRP * EB // 16, _zo, 0)

    for k in range(RPT // PIECE):
        pltpu.sync_copy(piece, acc.at[pl.ds(base_r + k * PIECE, PIECE)])
    pltpu.sync_copy(degbuf, dacc.at[pl.ds(base_r, RPT)])
    plsc.subcore_barrier()

    args = (acc, idxg2, idxs2, rows0, sem0, rows1, sem1, sid)

    @pl.when(c == 0)
    def _():
        _run_dir(x_hbm, srcg_hbm, dsts_hbm, *args, dacc=dacc, ones_v=ones_v)

    @pl.when(c == 1)
    def _():
        _run_dir(x_hbm, dstg_hbm, srcs_hbm, *args, dacc=dacc, ones_v=ones_v)

    plsc.subcore_barrier()

    def finish(s_hbm, d_hbm):
        pltpu.sync_copy(dacc.at[pl.ds(base_r, RPT)], degbuf)
        pltpu.sync_copy(degbuf, d_hbm.at[pl.ds(base_r, RPT)])
        for k in range(RPT // PIECE):
            pltpu.sync_copy(acc.at[pl.ds(base_r + k * PIECE, PIECE)], piece)
            pltpu.sync_copy(piece, s_hbm.at[pl.ds(base_r + k * PIECE, PIECE)])

    @pl.when(c == 0)
    def _():
        finish(sin_hbm, din_hbm)

    @pl.when(c == 1)
    def _():
        finish(sout_hbm, dout_hbm)


_phase1 = functools.partial(
    pl.kernel,
    out_type=[
        jax.ShapeDtypeStruct((NPAD, D), jnp.float32),
        jax.ShapeDtypeStruct((NPAD, D), jnp.float32),
        jax.ShapeDtypeStruct((NPAD,), jnp.float32),
        jax.ShapeDtypeStruct((NPAD,), jnp.float32),
    ],
    mesh=_sc_mesh(),
    scratch_types=[
        pltpu.VMEM((KGRP * EB,), jnp.int32),
        pltpu.VMEM((KGRP * EB,), jnp.int32),
        pltpu.VMEM((EB, D), jnp.float32),
        pltpu.VMEM((EB, D), jnp.float32),
        pltpu.VMEM((PIECE, D), jnp.float32),
        pltpu.VMEM((RPT,), jnp.float32),
        pltpu.VMEM((KGRP * EB,), jnp.float32),
        pltpu.VMEM_SHARED((NPAD, D), jnp.float32),
        pltpu.VMEM_SHARED((NPAD,), jnp.float32),
        pltpu.SemaphoreType.DMA,
        pltpu.SemaphoreType.DMA,
    ],
)(_phase1_body)


def _phase2_body(uin_hbm, uout_hbm, srcg_hbm, dstg_hbm, srcs_hbm, dsts_hbm,
                 tin_hbm, tout_hbm,
                 idxg2, idxs2, rows0, rows1, piece,
                 acc, sem0, sem1):
    c = lax.axis_index("c")
    sid = lax.axis_index("s")
    base_r = sid * RPT

    _zero_piece(piece)
    for k in range(RPT // PIECE):
        pltpu.sync_copy(piece, acc.at[pl.ds(base_r + k * PIECE, PIECE)])
    plsc.subcore_barrier()

    @pl.when(c == 0)
    def _():
        _run_dir(uin_hbm, srcg_hbm, dsts_hbm, acc, idxg2, idxs2, rows0,
                 sem0, rows1, sem1, sid)

    @pl.when(c == 1)
    def _():
        _run_dir(uout_hbm, dstg_hbm, srcs_hbm, acc, idxg2, idxs2, rows0,
                 sem0, rows1, sem1, sid)

    plsc.subcore_barrier()

    def finish(t_hbm):
        for k in range(RPT // PIECE):
            pltpu.sync_copy(acc.at[pl.ds(base_r + k * PIECE, PIECE)], piece)
            pltpu.sync_copy(piece, t_hbm.at[pl.ds(base_r + k * PIECE, PIECE)])

    @pl.when(c == 0)
    def _():
        finish(tin_hbm)

    @pl.when(c == 1)
    def _():
        finish(tout_hbm)


_phase2 = functools.partial(
    pl.kernel,
    out_type=[
        jax.ShapeDtypeStruct((NPAD, D), jnp.float32),
        jax.ShapeDtypeStruct((NPAD, D), jnp.float32),
    ],
    mesh=_sc_mesh(),
    scratch_types=[
        pltpu.VMEM((KGRP * EB,), jnp.int32),
        pltpu.VMEM((KGRP * EB,), jnp.int32),
        pltpu.VMEM((EB, D), jnp.float32),
        pltpu.VMEM((EB, D), jnp.float32),
        pltpu.VMEM((PIECE, D), jnp.float32),
        pltpu.VMEM_SHARED((NPAD, D), jnp.float32),
        pltpu.SemaphoreType.DMA,
        pltpu.SemaphoreType.DMA,
    ],
)(_phase2_body)


def _fold_body(W1_ref, W2_ref, b1c_ref, kin_ref, kout_ref, cin_ref, cout_ref):
    W1 = W1_ref[...]
    Wc = jnp.concatenate(
        [W1[:, :D] + W1[:, D:2 * D], W1[:, 2 * D:3 * D] + W1[:, 3 * D:]],
        axis=1)
    W2in = W2_ref[:, :1024]
    W2out = W2_ref[:, 1024:]
    b1c = b1c_ref[...]
    kin_ref[...] = jnp.dot(W2in, Wc, preferred_element_type=jnp.float32)
    kout_ref[...] = jnp.dot(W2out, Wc, preferred_element_type=jnp.float32)
    cin_ref[...] = jnp.dot(W2in, b1c, preferred_element_type=jnp.float32)
    cout_ref[...] = jnp.dot(W2out, b1c, preferred_element_type=jnp.float32)


def _fold(W1, W2, b1c):
    return pl.pallas_call(
        _fold_body,
        out_shape=[
            jax.ShapeDtypeStruct((D, 2 * D), jnp.float32),
            jax.ShapeDtypeStruct((D, 2 * D), jnp.float32),
            jax.ShapeDtypeStruct((D, 1), jnp.float32),
            jax.ShapeDtypeStruct((D, 1), jnp.float32),
        ],
    )(W1, W2, b1c)


BLK = 1024


def _elu(v):
    return jnp.where(v > 0.0, v, jnp.exp(v) - 1.0)


def _proj_body(sin_ref, sout_ref, din_ref, dout_ref, min_ref, mout_ref,
               cin_ref, cout_ref, uin_ref, uout_ref):
    inv_in = 1.0 / jnp.maximum(din_ref[...], 1.0)
    inv_out = 1.0 / jnp.maximum(dout_ref[...], 1.0)
    f = jnp.concatenate(
        [_elu(sin_ref[...] * inv_in), _elu(sout_ref[...] * inv_out)],
        axis=1)
    uin_ref[...] = (
        jnp.dot(f, min_ref[...], preferred_element_type=jnp.float32)
        + cin_ref[...])
    uout_ref[...] = (
        jnp.dot(f, mout_ref[...], preferred_element_type=jnp.float32)
        + cout_ref[...])


def _proj(s_in, s_out, din_c, dout_c, Min, Mout, cin_r, cout_r):
    grid = (NPAD // BLK,)
    row_spec = pl.BlockSpec((BLK, D), lambda i: (i, 0))
    col_spec = pl.BlockSpec((BLK, 1), lambda i: (i, 0))
    full_spec = pl.BlockSpec((2 * D, D), lambda i: (0, 0))
    bias_spec = pl.BlockSpec((1, D), lambda i: (0, 0))
    return pl.pallas_call(
        _proj_body,
        grid=grid,
        in_specs=[row_spec, row_spec, col_spec, col_spec,
                  full_spec, full_spec, bias_spec, bias_spec],
        out_specs=[row_spec, row_spec],
        out_shape=[
            jax.ShapeDtypeStruct((NPAD, D), jnp.float32),
            jax.ShapeDtypeStruct((NPAD, D), jnp.float32),
        ],
    )(s_in, s_out, din_c, dout_c, Min, Mout, cin_r, cout_r)


def _final_body(tin_ref, tout_ref, din_ref, dout_ref, b2_ref, out_ref):
    inv_in = 1.0 / jnp.maximum(din_ref[...], 1.0)
    inv_out = 1.0 / jnp.maximum(dout_ref[...], 1.0)
    out_ref[...] = (tin_ref[...] * inv_in + tout_ref[...] * inv_out
                    + b2_ref[...])


def _final(t_in, t_out, din_c, dout_c, b2r):
    grid = (NPAD // BLK,)
    row_spec = pl.BlockSpec((BLK, D), lambda i: (i, 0))
    col_spec = pl.BlockSpec((BLK, 1), lambda i: (i, 0))
    bias_spec = pl.BlockSpec((1, D), lambda i: (0, 0))
    return pl.pallas_call(
        _final_body,
        grid=grid,
        in_specs=[row_spec, row_spec, col_spec, col_spec, bias_spec],
        out_specs=row_spec,
        out_shape=jax.ShapeDtypeStruct((NPAD, D), jnp.float32),
    )(t_in, t_out, din_c, dout_c, b2r)


@jax.jit
def kernel(feature, edge_index, W1, b1, W2, b2):
    npad_e = EPAD - E
    pad_g = jnp.zeros((npad_e,), jnp.int32)
    pad_s = N + (jnp.arange(npad_e, dtype=jnp.int32) % (NPAD - N))
    srcg = jnp.concatenate([edge_index[0], pad_g])
    dstg = jnp.concatenate([edge_index[1], pad_g])
    srcs = jnp.concatenate([edge_index[0], pad_s])
    dsts = jnp.concatenate([edge_index[1], pad_s])
    b1c = b1.reshape(-1, 1)
    kin, kout, cin, cout = _fold(W1, W2, b1c)
    s_in, s_out, din, dout = _phase1(feature, srcg, dstg, srcs, dsts)
    din_c = din.reshape(NPAD, 1)
    dout_c = dout.reshape(NPAD, 1)
    u_in, u_out = _proj(s_in, s_out, din_c, dout_c,
                        kin.T, kout.T, cin.T, cout.T)
    t_in, t_out = _phase2(u_in, u_out, srcg, dstg, srcs, dsts)
    out = _final(t_in, t_out, din_c, dout_c, b2.reshape(1, -1))
    return out[:N]


# speedup vs baseline: 1.2763x; 1.0073x over previous
import functools

import jax
import jax.numpy as jnp
from jax import lax
from jax.experimental import pallas as pl
from jax.experimental.pallas import tpu as pltpu
from jax.experimental.pallas import tpu_sc as plsc

N = 10000
E = 320000
D = 128
NPAD = 10240
NTILE = 16
RPT = NPAD // NTILE
EB = 112
BPT = 180
KGRP = 10
EPAD = NTILE * BPT * EB
EPT = BPT * EB
PIECE = 32


def _sc_mesh():
    return plsc.VectorSubcoreMesh(core_axis_name="c", subcore_axis_name="s")


def _zero_piece(piece):
    z16 = jnp.zeros((16,), jnp.float32)

    def _zp(r, carry):
        for j in range(D // 16):
            piece[r, pl.ds(j * 16, 16)] = z16
        return carry
    lax.fori_loop(0, PIECE, _zp, 0)


def _run_dir(vals_hbm, g1d_hbm, s1d_hbm, acc, idxg2, idxs2,
             rows0, sem0, rows1, sem1, sid, dacc=None, ones_v=None):
    ebase = sid * EPT
    rows = (rows0, rows1)
    sems = (sem0, sem1)

    def group(g, carry):
        goff = ebase + g * (KGRP * EB)
        pltpu.sync_copy(g1d_hbm.at[pl.ds(goff, KGRP * EB)], idxg2)
        pltpu.sync_copy(s1d_hbm.at[pl.ds(goff, KGRP * EB)], idxs2)
        if dacc is not None:
            pltpu.sync_copy(ones_v, dacc.at[idxs2], add=True)
        descr = pltpu.async_copy(
            vals_hbm.at[idxg2.at[pl.ds(0, EB)]], rows[0], sems[0])
        for j in range(KGRP):
            nxt = None
            if j + 1 < KGRP:
                nxt = pltpu.async_copy(
                    vals_hbm.at[idxg2.at[pl.ds((j + 1) * EB, EB)]],
                    rows[(j + 1) % 2], sems[(j + 1) % 2])
            descr.wait()
            pltpu.sync_copy(rows[j % 2],
                            acc.at[idxs2.at[pl.ds(j * EB, EB)]], add=True)
            descr = nxt
        return carry
    lax.fori_loop(0, BPT // KGRP, group, 0)


def _phase1_body(x_hbm, srcg_hbm, dstg_hbm, srcs_hbm, dsts_hbm,
                 sin_hbm, sout_hbm, din_hbm, dout_hbm,
                 idxg2, idxs2, rows0, rows1, piece, degbuf, ones_v,
                 acc, dacc, sem0, sem1):
    c = lax.axis_index("c")
    sid = lax.axis_index("s")
    base_r = sid * RPT
    z16 = jnp.zeros((16,), jnp.float32)

    _zero_piece(piece)

    def _zd(i, carry):
        degbuf[pl.ds(i * 16, 16)] = z16
        return carry
    lax.fori_loop(0, RPT // 16, _zd, 0)

    def _zo(i, carry):
        ones_v[pl.ds(i * 16, 16)] = jnp.ones((16,), jnp.float32)
        return carry
    lax.fori_loop(0, KGRP * EB // 16, _zo, 0)

    for k in range(RPT // PIECE):
        pltpu.sync_copy(piece, acc.at[pl.ds(base_r + k * PIECE, PIECE)])
    pltpu.sync_copy(degbuf, dacc.at[pl.ds(base_r, RPT)])
    plsc.subcore_barrier()

    args = (acc, idxg2, idxs2, rows0, sem0, rows1, sem1, sid)

    @pl.when(c == 0)
    def _():
        _run_dir(x_hbm, srcg_hbm, dsts_hbm, *args, dacc=dacc, ones_v=ones_v)

    @pl.when(c == 1)
    def _():
        _run_dir(x_hbm, dstg_hbm, srcs_hbm, *args, dacc=dacc, ones_v=ones_v)

    plsc.subcore_barrier()

    def finish(s_hbm, d_hbm):
        pltpu.sync_copy(dacc.at[pl.ds(base_r, RPT)], d_hbm.at[pl.ds(base_r, RPT)])
        pltpu.sync_copy(acc.at[pl.ds(base_r, RPT)], s_hbm.at[pl.ds(base_r, RPT)])

    @pl.when(c == 0)
    def _():
        finish(sin_hbm, din_hbm)

    @pl.when(c == 1)
    def _():
        finish(sout_hbm, dout_hbm)


_phase1 = functools.partial(
    pl.kernel,
    out_type=[
        jax.ShapeDtypeStruct((NPAD, D), jnp.float32),
        jax.ShapeDtypeStruct((NPAD, D), jnp.float32),
        jax.ShapeDtypeStruct((NPAD,), jnp.float32),
        jax.ShapeDtypeStruct((NPAD,), jnp.float32),
    ],
    mesh=_sc_mesh(),
    scratch_types=[
        pltpu.VMEM((KGRP * EB,), jnp.int32),
        pltpu.VMEM((KGRP * EB,), jnp.int32),
        pltpu.VMEM((EB, D), jnp.float32),
        pltpu.VMEM((EB, D), jnp.float32),
        pltpu.VMEM((PIECE, D), jnp.float32),
        pltpu.VMEM((RPT,), jnp.float32),
        pltpu.VMEM((KGRP * EB,), jnp.float32),
        pltpu.VMEM_SHARED((NPAD, D), jnp.float32),
        pltpu.VMEM_SHARED((NPAD,), jnp.float32),
        pltpu.SemaphoreType.DMA,
        pltpu.SemaphoreType.DMA,
    ],
)(_phase1_body)


def _phase2_body(uin_hbm, uout_hbm, srcg_hbm, dstg_hbm, srcs_hbm, dsts_hbm,
                 tin_hbm, tout_hbm,
                 idxg2, idxs2, rows0, rows1, piece,
                 acc, sem0, sem1):
    c = lax.axis_index("c")
    sid = lax.axis_index("s")
    base_r = sid * RPT

    _zero_piece(piece)
    for k in range(RPT // PIECE):
        pltpu.sync_copy(piece, acc.at[pl.ds(base_r + k * PIECE, PIECE)])
    plsc.subcore_barrier()

    @pl.when(c == 0)
    def _():
        _run_dir(uin_hbm, srcg_hbm, dsts_hbm, acc, idxg2, idxs2, rows0,
                 sem0, rows1, sem1, sid)

    @pl.when(c == 1)
    def _():
        _run_dir(uout_hbm, dstg_hbm, srcs_hbm, acc, idxg2, idxs2, rows0,
                 sem0, rows1, sem1, sid)

    plsc.subcore_barrier()

    def finish(t_hbm):
        pltpu.sync_copy(acc.at[pl.ds(base_r, RPT)], t_hbm.at[pl.ds(base_r, RPT)])

    @pl.when(c == 0)
    def _():
        finish(tin_hbm)

    @pl.when(c == 1)
    def _():
        finish(tout_hbm)


_phase2 = functools.partial(
    pl.kernel,
    out_type=[
        jax.ShapeDtypeStruct((NPAD, D), jnp.float32),
        jax.ShapeDtypeStruct((NPAD, D), jnp.float32),
    ],
    mesh=_sc_mesh(),
    scratch_types=[
        pltpu.VMEM((KGRP * EB,), jnp.int32),
        pltpu.VMEM((KGRP * EB,), jnp.int32),
        pltpu.VMEM((EB, D), jnp.float32),
        pltpu.VMEM((EB, D), jnp.float32),
        pltpu.VMEM((PIECE, D), jnp.float32),
        pltpu.VMEM_SHARED((NPAD, D), jnp.float32),
        pltpu.SemaphoreType.DMA,
        pltpu.SemaphoreType.DMA,
    ],
)(_phase2_body)


def _fold_body(W1_ref, W2_ref, b1c_ref, kin_ref, kout_ref, cin_ref, cout_ref):
    W1 = W1_ref[...]
    Wc = jnp.concatenate(
        [W1[:, :D] + W1[:, D:2 * D], W1[:, 2 * D:3 * D] + W1[:, 3 * D:]],
        axis=1)
    W2in = W2_ref[:, :1024]
    W2out = W2_ref[:, 1024:]
    b1c = b1c_ref[...]
    kin_ref[...] = jnp.dot(W2in, Wc, preferred_element_type=jnp.float32)
    kout_ref[...] = jnp.dot(W2out, Wc, preferred_element_type=jnp.float32)
    cin_ref[...] = jnp.dot(W2in, b1c, preferred_element_type=jnp.float32)
    cout_ref[...] = jnp.dot(W2out, b1c, preferred_element_type=jnp.float32)


def _fold(W1, W2, b1c):
    return pl.pallas_call(
        _fold_body,
        out_shape=[
            jax.ShapeDtypeStruct((D, 2 * D), jnp.float32),
            jax.ShapeDtypeStruct((D, 2 * D), jnp.float32),
            jax.ShapeDtypeStruct((D, 1), jnp.float32),
            jax.ShapeDtypeStruct((D, 1), jnp.float32),
        ],
    )(W1, W2, b1c)


BLK = 1024


def _elu(v):
    return jnp.where(v > 0.0, v, jnp.exp(v) - 1.0)


def _proj_body(sin_ref, sout_ref, din_ref, dout_ref, min_ref, mout_ref,
               cin_ref, cout_ref, uin_ref, uout_ref):
    inv_in = 1.0 / jnp.maximum(din_ref[...], 1.0)
    inv_out = 1.0 / jnp.maximum(dout_ref[...], 1.0)
    f = jnp.concatenate(
        [_elu(sin_ref[...] * inv_in), _elu(sout_ref[...] * inv_out)],
        axis=1)
    uin_ref[...] = (
        jnp.dot(f, min_ref[...], preferred_element_type=jnp.float32)
        + cin_ref[...])
    uout_ref[...] = (
        jnp.dot(f, mout_ref[...], preferred_element_type=jnp.float32)
        + cout_ref[...])


def _proj(s_in, s_out, din_c, dout_c, Min, Mout, cin_r, cout_r):
    grid = (NPAD // BLK,)
    row_spec = pl.BlockSpec((BLK, D), lambda i: (i, 0))
    col_spec = pl.BlockSpec((BLK, 1), lambda i: (i, 0))
    full_spec = pl.BlockSpec((2 * D, D), lambda i: (0, 0))
    bias_spec = pl.BlockSpec((1, D), lambda i: (0, 0))
    return pl.pallas_call(
        _proj_body,
        grid=grid,
        in_specs=[row_spec, row_spec, col_spec, col_spec,
                  full_spec, full_spec, bias_spec, bias_spec],
        out_specs=[row_spec, row_spec],
        out_shape=[
            jax.ShapeDtypeStruct((NPAD, D), jnp.float32),
            jax.ShapeDtypeStruct((NPAD, D), jnp.float32),
        ],
    )(s_in, s_out, din_c, dout_c, Min, Mout, cin_r, cout_r)


def _final_body(tin_ref, tout_ref, din_ref, dout_ref, b2_ref, out_ref):
    inv_in = 1.0 / jnp.maximum(din_ref[...], 1.0)
    inv_out = 1.0 / jnp.maximum(dout_ref[...], 1.0)
    out_ref[...] = (tin_ref[...] * inv_in + tout_ref[...] * inv_out
                    + b2_ref[...])


def _final(t_in, t_out, din_c, dout_c, b2r):
    grid = (NPAD // BLK,)
    row_spec = pl.BlockSpec((BLK, D), lambda i: (i, 0))
    col_spec = pl.BlockSpec((BLK, 1), lambda i: (i, 0))
    bias_spec = pl.BlockSpec((1, D), lambda i: (0, 0))
    return pl.pallas_call(
        _final_body,
        grid=grid,
        in_specs=[row_spec, row_spec, col_spec, col_spec, bias_spec],
        out_specs=row_spec,
        out_shape=jax.ShapeDtypeStruct((NPAD, D), jnp.float32),
    )(t_in, t_out, din_c, dout_c, b2r)


@jax.jit
def kernel(feature, edge_index, W1, b1, W2, b2):
    npad_e = EPAD - E
    pad_g = jnp.zeros((npad_e,), jnp.int32)
    pad_s = N + (jnp.arange(npad_e, dtype=jnp.int32) % (NPAD - N))
    srcg = jnp.concatenate([edge_index[0], pad_g])
    dstg = jnp.concatenate([edge_index[1], pad_g])
    srcs = jnp.concatenate([edge_index[0], pad_s])
    dsts = jnp.concatenate([edge_index[1], pad_s])
    b1c = b1.reshape(-1, 1)
    kin, kout, cin, cout = _fold(W1, W2, b1c)
    s_in, s_out, din, dout = _phase1(feature, srcg, dstg, srcs, dsts)
    din_c = din.reshape(NPAD, 1)
    dout_c = dout.reshape(NPAD, 1)
    u_in, u_out = _proj(s_in, s_out, din_c, dout_c,
                        kin.T, kout.T, cin.T, cout.T)
    t_in, t_out = _phase2(u_in, u_out, srcg, dstg, srcs, dsts)
    out = _final(t_in, t_out, din_c, dout_c, b2.reshape(1, -1))
    return out[:N]
